# baseline (speedup 1.0000x reference)
import functools

import jax
import jax.numpy as jnp
from jax.experimental import pallas as pl
from jax.experimental.pallas import tpu as pltpu

N = 10000
IN_CH = 128
HID = 128
OUT_CH = 128
BLOCK_ROWS = 10000


def _fused_body(
    x_ref, wz_ref, wh_ref, bz_ref, bh_ref, lw_ref, lb_ref, out_ref, wzh_s, lw_s
):
    @pl.when(pl.program_id(0) == 0)
    def _prep():
        wzh_s[:, :HID] = (wz_ref[0, :IN_CH, :] + wz_ref[1, :IN_CH, :]).astype(
            jnp.bfloat16
        )
        wzh_s[:, HID:] = (wh_ref[0, :IN_CH, :] + wh_ref[1, :IN_CH, :]).astype(
            jnp.bfloat16
        )
        lw_s[...] = lw_ref[...].astype(jnp.bfloat16)

    xb = x_ref[...].astype(jnp.bfloat16)
    act = jnp.dot(xb, wzh_s[...], preferred_element_type=jnp.float32)
    z_bar = 0.5 - 0.5 * jnp.tanh(0.5 * (act[:, :HID] + bz_ref[...]))
    h = z_bar * jnp.tanh(act[:, HID:] + bh_ref[...])
    h = jnp.where(h > 0, h, jnp.exp(h) - 1.0)
    out_ref[...] = (
        jnp.dot(h.astype(jnp.bfloat16), lw_s[...], preferred_element_type=jnp.float32)
        + lb_ref[...]
    )


@functools.partial(jax.jit, static_argnames=())
def kernel(x, edge_index, edge_weight, W_z, b_z, W_r, b_r, W_h, b_h, lin_W, lin_b):
    del edge_index, edge_weight, W_r, b_r
    wz = W_z.reshape(2, IN_CH + HID, HID)
    wh = W_h.reshape(2, IN_CH + HID, HID)
    bz = b_z.reshape(1, HID)
    bh = b_h.reshape(1, HID)
    lb = lin_b.reshape(1, OUT_CH)

    grid = N // BLOCK_ROWS
    return pl.pallas_call(
        _fused_body,
        grid=(grid,),
        in_specs=[
            pl.BlockSpec((BLOCK_ROWS, IN_CH), lambda i: (i, 0)),
            pl.BlockSpec((2, IN_CH + HID, HID), lambda i: (0, 0, 0)),
            pl.BlockSpec((2, IN_CH + HID, HID), lambda i: (0, 0, 0)),
            pl.BlockSpec((1, HID), lambda i: (0, 0)),
            pl.BlockSpec((1, HID), lambda i: (0, 0)),
            pl.BlockSpec((HID, OUT_CH), lambda i: (0, 0)),
            pl.BlockSpec((1, OUT_CH), lambda i: (0, 0)),
        ],
        out_specs=pl.BlockSpec((BLOCK_ROWS, OUT_CH), lambda i: (i, 0)),
        out_shape=jax.ShapeDtypeStruct((N, OUT_CH), x.dtype),
        scratch_shapes=[
            pltpu.VMEM((IN_CH, 2 * HID), jnp.bfloat16),
            pltpu.VMEM((HID, OUT_CH), jnp.bfloat16),
        ],
    )(x, wz, wh, bz, bh, lin_W, lb)


# speedup vs baseline: 1.1411x; 1.1411x over previous
import functools

import jax
import jax.numpy as jnp
from jax.experimental import pallas as pl
from jax.experimental.pallas import tpu as pltpu

N = 10000
IN_CH = 128
HID = 128
OUT_CH = 128
BLOCK_ROWS = 5000


def _fused_body(
    x_ref, wz_ref, wh_ref, bz_ref, bh_ref, lw_ref, lb_ref, out_ref, wzh_s, lw_s
):
    @pl.when(pl.program_id(0) == 0)
    def _prep():
        wzh_s[:, :HID] = (wz_ref[0, :IN_CH, :] + wz_ref[1, :IN_CH, :]).astype(
            jnp.bfloat16
        )
        wzh_s[:, HID:] = (wh_ref[0, :IN_CH, :] + wh_ref[1, :IN_CH, :]).astype(
            jnp.bfloat16
        )
        lw_s[...] = lw_ref[...].astype(jnp.bfloat16)

    one = jnp.bfloat16(1.0)
    half = jnp.bfloat16(0.5)
    xb = x_ref[...].astype(jnp.bfloat16)
    act = jnp.dot(xb, wzh_s[...], preferred_element_type=jnp.float32).astype(
        jnp.bfloat16
    )
    bz = bz_ref[...].astype(jnp.bfloat16)
    bh = bh_ref[...].astype(jnp.bfloat16)
    z_bar = half - half * jnp.tanh(half * (act[:, :HID] + bz))
    h = z_bar * jnp.tanh(act[:, HID:] + bh)
    h = jnp.where(h > 0, h, jnp.exp(h) - one)
    out_ref[...] = (
        jnp.dot(h, lw_s[...], preferred_element_type=jnp.float32) + lb_ref[...]
    )


@functools.partial(jax.jit, static_argnames=())
def kernel(x, edge_index, edge_weight, W_z, b_z, W_r, b_r, W_h, b_h, lin_W, lin_b):
    del edge_index, edge_weight, W_r, b_r
    wz = W_z.reshape(2, IN_CH + HID, HID)
    wh = W_h.reshape(2, IN_CH + HID, HID)
    bz = b_z.reshape(1, HID)
    bh = b_h.reshape(1, HID)
    lb = lin_b.reshape(1, OUT_CH)

    grid = N // BLOCK_ROWS
    return pl.pallas_call(
        _fused_body,
        grid=(grid,),
        in_specs=[
            pl.BlockSpec((BLOCK_ROWS, IN_CH), lambda i: (i, 0)),
            pl.BlockSpec((2, IN_CH + HID, HID), lambda i: (0, 0, 0)),
            pl.BlockSpec((2, IN_CH + HID, HID), lambda i: (0, 0, 0)),
            pl.BlockSpec((1, HID), lambda i: (0, 0)),
            pl.BlockSpec((1, HID), lambda i: (0, 0)),
            pl.BlockSpec((HID, OUT_CH), lambda i: (0, 0)),
            pl.BlockSpec((1, OUT_CH), lambda i: (0, 0)),
        ],
        out_specs=pl.BlockSpec((BLOCK_ROWS, OUT_CH), lambda i: (i, 0)),
        out_shape=jax.ShapeDtypeStruct((N, OUT_CH), x.dtype),
        scratch_shapes=[
            pltpu.VMEM((IN_CH, 2 * HID), jnp.bfloat16),
            pltpu.VMEM((HID, OUT_CH), jnp.bfloat16),
        ],
    )(x, wz, wh, bz, bh, lin_W, lb)
